# Initial kernel scaffold; baseline (speedup 1.0000x reference)
import jax
import jax.numpy as jnp
from jax.experimental import pallas as pl


def kernel(x, edge_index, Wx, bx, Wh, bh, wc, b, ln_gamma, ln_beta):
    raise NotImplementedError("write your pallas kernel here")


# speedup vs baseline: 17.3838x; 17.3838x over previous
import functools

import jax
import jax.numpy as jnp
from jax import lax
from jax.experimental import pallas as pl
from jax.experimental.pallas import tpu as pltpu
from jax.experimental.pallas import tpu_sc as plsc

N = 10000
E = 640000
CIN = 16
D = 64
T = 4

NC, NS, L = 2, 16, 16
NW = NC * NS
NPAD = 10240
RPT = NPAD // NS
TRASH = N
SUB = 128
NSUB = 4
CH = SUB * NSUB
NCHUNK = 40
EPT = CH * NCHUNK
EPAD = EPT * NW
E128 = EPAD // 128
CPW = EPT // 128

_mesh = plsc.VectorSubcoreMesh(
    core_axis_name="c", subcore_axis_name="s", num_cores=NC, num_subcores=NS)


def _spmv_body(y_hbm, row_hbm, col_hbm, out_hbm, rbuf, cbuf, gat, zbuf, acc, sem):
    c = lax.axis_index("c")
    s = lax.axis_index("s")
    wid = c * NS + s
    zero = jnp.zeros((L,), jnp.float32)
    for i in range(16):
        for j in range(D // L):
            zbuf[i, j * L:(j + 1) * L] = zero

    def zbody(k, carry):
        pltpu.sync_copy(zbuf, acc.at[pl.ds(s * RPT + k * 16, 16)])
        return carry

    lax.fori_loop(0, RPT // 16, zbody, 0)
    plsc.subcore_barrier()

    def cbody(ci, carry):
        b = wid * CPW + ci * NSUB
        pltpu.sync_copy(row_hbm.at[pl.ds(b, NSUB)], rbuf)
        pltpu.sync_copy(col_hbm.at[pl.ds(b, NSUB)], cbuf)
        hs = [pltpu.async_copy(y_hbm.at[rbuf.at[j]], gat.at[j], sem)
              for j in range(NSUB)]
        for h in hs:
            h.wait()
        for j in range(NSUB):
            pltpu.sync_copy(gat.at[j], acc.at[cbuf.at[j]], add=True)
        return carry

    lax.fori_loop(0, NCHUNK, cbody, 0)
    plsc.subcore_barrier()
    pltpu.sync_copy(acc.at[pl.ds(s * RPT, RPT)], out_hbm.at[c, pl.ds(s * RPT, RPT)])


_spmv = pl.kernel(
    _spmv_body,
    out_type=jax.ShapeDtypeStruct((NC, NPAD, D), jnp.float32),
    mesh=_mesh,
    scratch_types=[
        pltpu.VMEM((NSUB, SUB), jnp.int32),
        pltpu.VMEM((NSUB, SUB), jnp.int32),
        pltpu.VMEM((NSUB, SUB, D), jnp.float32),
        pltpu.VMEM((16, D), jnp.float32),
        pltpu.VMEM_SHARED((NPAD, D), jnp.float32),
        pltpu.SemaphoreType.DMA,
    ],
    compiler_params=pltpu.CompilerParams(use_tc_tiling_on_sc=False),
)

W16 = 16


def _deg_body(rowd_hbm, out_hbm, rbuf, obuf, zbuf, acc):
    c = lax.axis_index("c")
    s = lax.axis_index("s")
    wid = c * NS + s
    zero = jnp.zeros((L,), jnp.float32)
    one = jnp.ones((L,), jnp.float32)
    for i in range(16):
        zbuf[i, :] = zero
    for i in range(SUB):
        obuf[i, :] = one

    def zbody(k, carry):
        pltpu.sync_copy(zbuf, acc.at[pl.ds(s * RPT + k * 16, 16)])
        return carry

    lax.fori_loop(0, RPT // 16, zbody, 0)
    plsc.subcore_barrier()

    def cbody(ci, carry):
        b = wid * CPW + ci * NSUB
        pltpu.sync_copy(rowd_hbm.at[pl.ds(b, NSUB)], rbuf)
        for j in range(NSUB):
            pltpu.sync_copy(obuf, acc.at[rbuf.at[j]], add=True)
        return carry

    lax.fori_loop(0, NCHUNK, cbody, 0)
    plsc.subcore_barrier()
    pltpu.sync_copy(acc.at[pl.ds(s * RPT, RPT)], out_hbm.at[c, pl.ds(s * RPT, RPT)])


_deg = pl.kernel(
    _deg_body,
    out_type=jax.ShapeDtypeStruct((NC, NPAD, W16), jnp.float32),
    mesh=_mesh,
    scratch_types=[
        pltpu.VMEM((NSUB, SUB), jnp.int32),
        pltpu.VMEM((SUB, W16), jnp.float32),
        pltpu.VMEM((16, W16), jnp.float32),
        pltpu.VMEM_SHARED((NPAD, W16), jnp.float32),
    ],
    compiler_params=pltpu.CompilerParams(use_tc_tiling_on_sc=False),
)

RB = 1280
GRID = NPAD // RB


def _prep_body(degp_ref, xtf_ref, dinv_ref, yx_ref):
    deg = degp_ref[0, :, 0:1] + degp_ref[1, :, 0:1]
    dinv = jnp.where(deg > 0, lax.rsqrt(deg), 0.0)
    dinv_ref[...] = jnp.broadcast_to(dinv, (RB, D))
    yx_ref[...] = dinv * xtf_ref[...]


_prep = pl.pallas_call(
    _prep_body,
    grid=(GRID,),
    in_specs=[
        pl.BlockSpec((NC, RB, W16), lambda i: (0, i, 0)),
        pl.BlockSpec((RB, T * CIN), lambda i: (i, 0)),
    ],
    out_specs=[pl.BlockSpec((RB, D), lambda i: (i, 0))] * 2,
    out_shape=[jax.ShapeDtypeStruct((NPAD, D), jnp.float32)] * 2,
)


def _step_body(xt_ref, tzxp_ref, hzp_ref, h_ref, c_ref, dv_ref,
               w0x_ref, w1x_ref, w0h_ref, w1h_ref, bc_ref, misc_ref,
               ho_ref, co_ref, yo_ref, no_ref):
    dv = dv_ref[...]
    xt = xt_ref[...]
    tzx = -dv[:, :CIN] * (tzxp_ref[0] + tzxp_ref[1])
    h = h_ref[...]
    tzh = -dv * (hzp_ref[0] + hzp_ref[1])
    g = (jnp.dot(xt, w0x_ref[...], preferred_element_type=jnp.float32)
         + jnp.dot(tzx, w1x_ref[...], preferred_element_type=jnp.float32)
         + jnp.dot(h, w0h_ref[...], preferred_element_type=jnp.float32)
         + jnp.dot(tzh, w1h_ref[...], preferred_element_type=jnp.float32)
         + bc_ref[...])
    cc = c_ref[...]
    wc0 = misc_ref[0:1, :]
    wc1 = misc_ref[1:2, :]
    wc2 = misc_ref[2:3, :]
    gamma = misc_ref[4:5, :]
    beta = misc_ref[5:6, :]
    ig = jax.nn.sigmoid(g[:, 0:D] + wc0 * cc)
    fg = jax.nn.sigmoid(g[:, D:2 * D] + wc1 * cc)
    tg = jnp.tanh(g[:, 2 * D:3 * D])
    cn = fg * cc + ig * tg
    og = jax.nn.sigmoid(g[:, 3 * D:4 * D] + wc2 * cn)
    hn = og * jnp.tanh(cn)
    ho_ref[...] = hn
    co_ref[...] = cn
    yo_ref[...] = dv * hn
    th = jnp.tanh(hn)
    mu = jnp.mean(th, axis=1, keepdims=True)
    var = jnp.mean((th - mu) ** 2, axis=1, keepdims=True)
    no_ref[...] = (th - mu) * lax.rsqrt(var + 1e-5) * gamma + beta


_step = pl.pallas_call(
    _step_body,
    grid=(GRID,),
    in_specs=[
        pl.BlockSpec((RB, CIN), lambda i: (i, 0)),
        pl.BlockSpec((NC, RB, CIN), lambda i: (0, i, 0)),
        pl.BlockSpec((NC, RB, D), lambda i: (0, i, 0)),
        pl.BlockSpec((RB, D), lambda i: (i, 0)),
        pl.BlockSpec((RB, D), lambda i: (i, 0)),
        pl.BlockSpec((RB, D), lambda i: (i, 0)),
        pl.BlockSpec((CIN, 4 * D), lambda i: (0, 0)),
        pl.BlockSpec((CIN, 4 * D), lambda i: (0, 0)),
        pl.BlockSpec((D, 4 * D), lambda i: (0, 0)),
        pl.BlockSpec((D, 4 * D), lambda i: (0, 0)),
        pl.BlockSpec((1, 4 * D), lambda i: (0, 0)),
        pl.BlockSpec((8, D), lambda i: (0, 0)),
    ],
    out_specs=[pl.BlockSpec((RB, D), lambda i: (i, 0))] * 4,
    out_shape=[jax.ShapeDtypeStruct((NPAD, D), jnp.float32)] * 4,
)


def kernel(x, edge_index, Wx, bx, Wh, bh, wc, b, ln_gamma, ln_beta):
    row = edge_index[0].astype(jnp.int32)
    col = edge_index[1].astype(jnp.int32)
    self_m = row == col
    col_s = jnp.where(self_m, TRASH, col)
    rowd = jnp.where(self_m, TRASH, row)
    pad = EPAD - E
    row_p = jnp.concatenate(
        [row, jnp.zeros((pad,), jnp.int32)]).reshape(E128, 128)
    col_p = jnp.concatenate(
        [col_s, jnp.full((pad,), TRASH, jnp.int32)]).reshape(E128, 128)
    rowd_p = jnp.concatenate(
        [rowd, jnp.full((pad,), TRASH, jnp.int32)]).reshape(E128, 128)

    xtf = jnp.pad(x.transpose(0, 2, 1).reshape(N, T * CIN),
                  ((0, NPAD - N), (0, 0)))

    degp = _deg(rowd_p)
    dinv2d, yx = _prep(degp, xtf)
    tzxp = _spmv(yx, row_p, col_p)

    w0x = Wx[:, 0].transpose(1, 0, 2).reshape(CIN, 4 * D)
    w1x = Wx[:, 1].transpose(1, 0, 2).reshape(CIN, 4 * D)
    w0h = Wh[:, 0].transpose(1, 0, 2).reshape(D, 4 * D)
    w1h = Wh[:, 1].transpose(1, 0, 2).reshape(D, 4 * D)
    bc = (bx + bh + b).transpose(0, 1).reshape(1, 4 * D)
    misc = (jnp.zeros((8, D), jnp.float32)
            .at[0:3].set(wc).at[4].set(ln_gamma).at[5].set(ln_beta))

    h = jnp.zeros((NPAD, D), jnp.float32)
    c = jnp.zeros((NPAD, D), jnp.float32)
    hzp = jnp.zeros((NC, NPAD, D), jnp.float32)
    outs = []
    for t in range(T):
        xt = lax.slice_in_dim(xtf, t * CIN, (t + 1) * CIN, axis=1)
        tzxp_t = lax.slice_in_dim(tzxp, t * CIN, (t + 1) * CIN, axis=2)
        h, c, yh, hn = _step(xt, tzxp_t, hzp, h, c, dinv2d,
                             w0x, w1x, w0h, w1h, bc, misc)
        outs.append(hn[:N])
        if t < T - 1:
            hzp = _spmv(yh, row_p, col_p)
    return jnp.stack(outs, axis=0)


# speedup vs baseline: 19.6155x; 1.1284x over previous
import functools

import jax
import jax.numpy as jnp
from jax import lax
from jax.experimental import pallas as pl
from jax.experimental.pallas import tpu as pltpu
from jax.experimental.pallas import tpu_sc as plsc

N = 10000
E = 640000
CIN = 16
D = 64
T = 4

NC, NS, L = 2, 16, 16
NW = NC * NS
NPAD = 10240
RPT = NPAD // NS
TRASH = N
SUB = 128
NSUB = 4
GROUP = 5
NG = 32
NIB = 4
NCHUNK = GROUP * NG
EPT = SUB * NCHUNK
EPAD = EPT * NW
E128 = EPAD // 128
CPW = EPT // 128

_mesh = plsc.VectorSubcoreMesh(
    core_axis_name="c", subcore_axis_name="s", num_cores=NC, num_subcores=NS)


def _spmv_body(y_hbm, rc_hbm, out_hbm, ibuf, gat, zbuf, acc, *sems):
    isem = sems[0:NIB]
    gsem = sems[NIB:NIB + GROUP]
    ssem = sems[NIB + GROUP:NIB + 2 * GROUP]
    c = lax.axis_index("c")
    s = lax.axis_index("s")
    wid = c * NS + s
    ibase = wid * CPW
    zero = jnp.zeros((L,), jnp.float32)
    for i in range(16):
        for j in range(D // L):
            zbuf[i, j * L:(j + 1) * L] = zero
    for g0 in range(2):
        pltpu.async_copy(rc_hbm.at[pl.ds(ibase + g0 * GROUP, GROUP)],
                         ibuf.at[g0], isem[g0])

    def zbody(k, carry):
        pltpu.sync_copy(zbuf, acc.at[pl.ds(s * RPT + k * 16, 16)])
        return carry

    lax.fori_loop(0, RPT // 16, zbody, 0)
    plsc.subcore_barrier()

    def gbody(gg, carry):
        for q in range(NIB):
            g = gg * NIB + q
            s2 = (q + 2) % NIB

            def fire_idx():
                pltpu.async_copy(rc_hbm.at[pl.ds(ibase + (g + 2) * GROUP, GROUP)],
                                 ibuf.at[s2], isem[s2])

            if q < 2:
                fire_idx()
            else:
                pl.when(gg < NG // NIB - 1)(fire_idx)
            pltpu.make_async_copy(rc_hbm.at[pl.ds(ibase + g * GROUP, GROUP)],
                                  ibuf.at[q], isem[q]).wait()
            for b in range(GROUP):
                def wait_sc(qp=(q - 1) % NIB, b=b):
                    pltpu.make_async_copy(gat.at[b], acc.at[ibuf.at[qp, b, 1]],
                                          ssem[b]).wait()

                if q == 0:
                    pl.when(gg > 0)(wait_sc)
                else:
                    wait_sc()
                pltpu.async_copy(y_hbm.at[ibuf.at[q, b, 0]], gat.at[b], gsem[b])
            for b in range(GROUP):
                pltpu.make_async_copy(y_hbm.at[ibuf.at[q, b, 0]], gat.at[b],
                                      gsem[b]).wait()
                pltpu.async_copy(gat.at[b], acc.at[ibuf.at[q, b, 1]], ssem[b],
                                 add=True)
        return carry

    lax.fori_loop(0, NG // NIB, gbody, 0)
    for b in range(GROUP):
        pltpu.make_async_copy(gat.at[b], acc.at[ibuf.at[NIB - 1, b, 1]],
                              ssem[b]).wait()
    plsc.subcore_barrier()
    pltpu.sync_copy(acc.at[pl.ds(s * RPT, RPT)], out_hbm.at[c, pl.ds(s * RPT, RPT)])


_spmv = pl.kernel(
    _spmv_body,
    out_type=jax.ShapeDtypeStruct((NC, NPAD, D), jnp.float32),
    mesh=_mesh,
    scratch_types=[
        pltpu.VMEM((NIB, GROUP, 2, SUB), jnp.int32),
        pltpu.VMEM((GROUP, SUB, D), jnp.float32),
        pltpu.VMEM((16, D), jnp.float32),
        pltpu.VMEM_SHARED((NPAD, D), jnp.float32),
    ] + [pltpu.SemaphoreType.DMA] * (NIB + 2 * GROUP),
    compiler_params=pltpu.CompilerParams(use_tc_tiling_on_sc=False),
)

W16 = 16


def _deg_body(rowd_hbm, out_hbm, rbuf, obuf, zbuf, acc):
    c = lax.axis_index("c")
    s = lax.axis_index("s")
    wid = c * NS + s
    zero = jnp.zeros((L,), jnp.float32)
    one = jnp.ones((L,), jnp.float32)
    for i in range(16):
        zbuf[i, :] = zero
    for i in range(SUB):
        obuf[i, :] = one

    def zbody(k, carry):
        pltpu.sync_copy(zbuf, acc.at[pl.ds(s * RPT + k * 16, 16)])
        return carry

    lax.fori_loop(0, RPT // 16, zbody, 0)
    plsc.subcore_barrier()

    def cbody(ci, carry):
        b = wid * CPW + ci * NSUB
        pltpu.sync_copy(rowd_hbm.at[pl.ds(b, NSUB)], rbuf)
        for j in range(NSUB):
            pltpu.sync_copy(obuf, acc.at[rbuf.at[j]], add=True)
        return carry

    lax.fori_loop(0, CPW // NSUB, cbody, 0)
    plsc.subcore_barrier()
    pltpu.sync_copy(acc.at[pl.ds(s * RPT, RPT)], out_hbm.at[c, pl.ds(s * RPT, RPT)])


_deg = pl.kernel(
    _deg_body,
    out_type=jax.ShapeDtypeStruct((NC, NPAD, W16), jnp.float32),
    mesh=_mesh,
    scratch_types=[
        pltpu.VMEM((NSUB, SUB), jnp.int32),
        pltpu.VMEM((SUB, W16), jnp.float32),
        pltpu.VMEM((16, W16), jnp.float32),
        pltpu.VMEM_SHARED((NPAD, W16), jnp.float32),
    ],
    compiler_params=pltpu.CompilerParams(use_tc_tiling_on_sc=False),
)

RB = 1280
GRID = NPAD // RB


def _prep_body(degp_ref, xtf_ref, dinv_ref, yx_ref):
    deg = degp_ref[0, :, 0:1] + degp_ref[1, :, 0:1]
    dinv = jnp.where(deg > 0, lax.rsqrt(deg), 0.0)
    dinv_ref[...] = jnp.broadcast_to(dinv, (RB, D))
    yx_ref[...] = dinv * xtf_ref[...]


_prep = pl.pallas_call(
    _prep_body,
    grid=(GRID,),
    in_specs=[
        pl.BlockSpec((NC, RB, W16), lambda i: (0, i, 0)),
        pl.BlockSpec((RB, T * CIN), lambda i: (i, 0)),
    ],
    out_specs=[pl.BlockSpec((RB, D), lambda i: (i, 0))] * 2,
    out_shape=[jax.ShapeDtypeStruct((NPAD, D), jnp.float32)] * 2,
)


def _step_body(xt_ref, tzxp_ref, hzp_ref, h_ref, c_ref, dv_ref,
               w0x_ref, w1x_ref, w0h_ref, w1h_ref, bc_ref, misc_ref,
               ho_ref, co_ref, yo_ref, no_ref):
    dv = dv_ref[...]
    xt = xt_ref[...]
    tzx = -dv[:, :CIN] * (tzxp_ref[0] + tzxp_ref[1])
    h = h_ref[...]
    tzh = -dv * (hzp_ref[0] + hzp_ref[1])
    g = (jnp.dot(xt, w0x_ref[...], preferred_element_type=jnp.float32)
         + jnp.dot(tzx, w1x_ref[...], preferred_element_type=jnp.float32)
         + jnp.dot(h, w0h_ref[...], preferred_element_type=jnp.float32)
         + jnp.dot(tzh, w1h_ref[...], preferred_element_type=jnp.float32)
         + bc_ref[...])
    cc = c_ref[...]
    wc0 = misc_ref[0:1, :]
    wc1 = misc_ref[1:2, :]
    wc2 = misc_ref[2:3, :]
    gamma = misc_ref[4:5, :]
    beta = misc_ref[5:6, :]
    ig = jax.nn.sigmoid(g[:, 0:D] + wc0 * cc)
    fg = jax.nn.sigmoid(g[:, D:2 * D] + wc1 * cc)
    tg = jnp.tanh(g[:, 2 * D:3 * D])
    cn = fg * cc + ig * tg
    og = jax.nn.sigmoid(g[:, 3 * D:4 * D] + wc2 * cn)
    hn = og * jnp.tanh(cn)
    ho_ref[...] = hn
    co_ref[...] = cn
    yo_ref[...] = dv * hn
    th = jnp.tanh(hn)
    mu = jnp.mean(th, axis=1, keepdims=True)
    var = jnp.mean((th - mu) ** 2, axis=1, keepdims=True)
    no_ref[...] = (th - mu) * lax.rsqrt(var + 1e-5) * gamma + beta


_step = pl.pallas_call(
    _step_body,
    grid=(GRID,),
    in_specs=[
        pl.BlockSpec((RB, CIN), lambda i: (i, 0)),
        pl.BlockSpec((NC, RB, CIN), lambda i: (0, i, 0)),
        pl.BlockSpec((NC, RB, D), lambda i: (0, i, 0)),
        pl.BlockSpec((RB, D), lambda i: (i, 0)),
        pl.BlockSpec((RB, D), lambda i: (i, 0)),
        pl.BlockSpec((RB, D), lambda i: (i, 0)),
        pl.BlockSpec((CIN, 4 * D), lambda i: (0, 0)),
        pl.BlockSpec((CIN, 4 * D), lambda i: (0, 0)),
        pl.BlockSpec((D, 4 * D), lambda i: (0, 0)),
        pl.BlockSpec((D, 4 * D), lambda i: (0, 0)),
        pl.BlockSpec((1, 4 * D), lambda i: (0, 0)),
        pl.BlockSpec((8, D), lambda i: (0, 0)),
    ],
    out_specs=[pl.BlockSpec((RB, D), lambda i: (i, 0))] * 4,
    out_shape=[jax.ShapeDtypeStruct((NPAD, D), jnp.float32)] * 4,
)


def kernel(x, edge_index, Wx, bx, Wh, bh, wc, b, ln_gamma, ln_beta):
    row = edge_index[0].astype(jnp.int32)
    col = edge_index[1].astype(jnp.int32)
    self_m = row == col
    col_s = jnp.where(self_m, TRASH, col)
    rowd = jnp.where(self_m, TRASH, row)
    pad = EPAD - E
    row_p = jnp.concatenate(
        [row, jnp.zeros((pad,), jnp.int32)]).reshape(E128, 128)
    col_p = jnp.concatenate(
        [col_s, jnp.full((pad,), TRASH, jnp.int32)]).reshape(E128, 128)
    rc = jnp.stack([row_p, col_p], axis=1)
    rowd_p = jnp.concatenate(
        [rowd, jnp.full((pad,), TRASH, jnp.int32)]).reshape(E128, 128)

    xtf = jnp.pad(x.transpose(0, 2, 1).reshape(N, T * CIN),
                  ((0, NPAD - N), (0, 0)))

    degp = _deg(rowd_p)
    dinv2d, yx = _prep(degp, xtf)
    tzxp = _spmv(yx, rc)

    w0x = Wx[:, 0].transpose(1, 0, 2).reshape(CIN, 4 * D)
    w1x = Wx[:, 1].transpose(1, 0, 2).reshape(CIN, 4 * D)
    w0h = Wh[:, 0].transpose(1, 0, 2).reshape(D, 4 * D)
    w1h = Wh[:, 1].transpose(1, 0, 2).reshape(D, 4 * D)
    bc = (bx + bh + b).transpose(0, 1).reshape(1, 4 * D)
    misc = (jnp.zeros((8, D), jnp.float32)
            .at[0:3].set(wc).at[4].set(ln_gamma).at[5].set(ln_beta))

    h = jnp.zeros((NPAD, D), jnp.float32)
    c = jnp.zeros((NPAD, D), jnp.float32)
    hzp = jnp.zeros((NC, NPAD, D), jnp.float32)
    outs = []
    for t in range(T):
        xt = lax.slice_in_dim(xtf, t * CIN, (t + 1) * CIN, axis=1)
        tzxp_t = lax.slice_in_dim(tzxp, t * CIN, (t + 1) * CIN, axis=2)
        h, c, yh, hn = _step(xt, tzxp_t, hzp, h, c, dinv2d,
                             w0x, w1x, w0h, w1h, bc, misc)
        outs.append(hn[:N])
        if t < T - 1:
            hzp = _spmv(yh, rc)
    return jnp.stack(outs, axis=0)
